# baseline (speedup 1.0000x reference)
import functools

import numpy as np

import jax
import jax.numpy as jnp
from jax import lax
from jax.experimental import pallas as pl
from jax.experimental.pallas import tpu as pltpu
from jax.experimental.pallas import tpu_sc as plsc

B, L, D = 4096, 50, 64
BL = B * L
NC, NS = 2, 16
NW = NC * NS
PER_W = BL // NW
CH = 128
NCH = PER_W // CH
RING = 5

_AR = np.arange(BL, dtype=np.int32)
_OFFS = ((_AR % L) * B + _AR // L).reshape(NW, NCH, CH)

_MESH = plsc.VectorSubcoreMesh(core_axis_name="c", subcore_axis_name="s")


@functools.partial(
    pl.kernel,
    mesh=_MESH,
    out_type=jax.ShapeDtypeStruct((BL, 128), jnp.float32),
    scratch_types=[
        pltpu.VMEM((NCH, CH), jnp.int32),
        pltpu.VMEM((NCH, CH), jnp.int32),
        pltpu.VMEM((RING, CH, 128), jnp.float32),
    ]
    + [pltpu.SemaphoreType.DMA] * (2 * RING),
)
def _gather_sc(idx_hbm, offs_hbm, table_hbm, g2_hbm, idx_v, offs_v, rows_v,
               *sems):
    gsems, ssems = sems[:RING], sems[RING:]
    wid = lax.axis_index("s") * NC + lax.axis_index("c")
    pltpu.sync_copy(idx_hbm.at[wid], idx_v)
    pltpu.sync_copy(offs_hbm.at[wid], offs_v)
    for b in range(RING):
        pltpu.async_copy(table_hbm.at[idx_v.at[b]], rows_v.at[b], gsems[b])

    def body(i, carry):
        for b in range(RING):
            j = i * RING + b
            pltpu.make_async_copy(
                table_hbm.at[idx_v.at[j]], rows_v.at[b], gsems[b]
            ).wait()
            pltpu.async_copy(rows_v.at[b], g2_hbm.at[offs_v.at[j]], ssems[b])

            @pl.when(i < NCH // RING - 1)
            def _():
                pltpu.make_async_copy(
                    rows_v.at[b], g2_hbm.at[offs_v.at[j]], ssems[b]
                ).wait()
                pltpu.async_copy(
                    table_hbm.at[idx_v.at[j + RING]], rows_v.at[b], gsems[b]
                )

        return carry

    lax.fori_loop(0, NCH // RING, body, 0)
    for b in range(RING):
        pltpu.make_async_copy(
            rows_v.at[b], g2_hbm.at[offs_v.at[NCH - RING + b]], ssems[b]
        ).wait()


CW = 2048
NT = 245
HALF = NT * CW


def _relayout_body(lo_ref, hi_ref, o_ref):
    ii = lax.broadcasted_iota(jnp.int32, (D, D), 0)
    jj = lax.broadcasted_iota(jnp.int32, (D, D), 1)
    ident = (ii == jj).astype(jnp.float32)
    dn = (((0,), (0,)), ((), ()))
    t0 = lax.dot_general(lo_ref[...], ident, dn,
                         precision=lax.Precision.HIGHEST,
                         preferred_element_type=jnp.float32)
    t1 = lax.dot_general(hi_ref[...], ident, dn,
                         precision=lax.Precision.HIGHEST,
                         preferred_element_type=jnp.float32)
    o_ref[...] = jnp.concatenate([t0, t1], axis=1)[None]


def _relayout_tc(table_t):
    out = pl.pallas_call(
        _relayout_body,
        grid=(NT,),
        in_specs=[
            pl.BlockSpec((D, CW), lambda i: (0, i)),
            pl.BlockSpec((D, CW), lambda i: (0, jnp.minimum(NT + i, 2 * NT - 2))),
        ],
        out_specs=pl.BlockSpec((1, CW, 128), lambda i: (i, 0, 0)),
        out_shape=jax.ShapeDtypeStruct((NT, CW, 128), jnp.float32),
    )(table_t, table_t)
    return out.reshape(HALF, 128)


NB = 4096


def _sel_tr_body(g_ref, par_ref, o_ref):
    gt = jnp.swapaxes(g_ref[...], 1, 2)
    par = par_ref[...]
    o_ref[...] = jnp.where(par == 1, gt[:, D:, :], gt[:, :D, :])


def _select_transpose_tc(g3, par):
    return pl.pallas_call(
        _sel_tr_body,
        grid=(L, B // NB),
        in_specs=[
            pl.BlockSpec((1, NB, 128), lambda l, i: (l, i, 0)),
            pl.BlockSpec((1, 1, NB), lambda l, i: (l, 0, i)),
        ],
        out_specs=pl.BlockSpec((1, D, NB), lambda l, i: (l, 0, i)),
        out_shape=jax.ShapeDtypeStruct((L, D, B), jnp.float32),
    )(g3, par)


def kernel(x, table):
    xi = x.astype(jnp.int32)
    idx2 = jnp.where(xi < HALF, xi, xi - HALF).reshape(NW, NCH, CH)
    offs = jnp.asarray(_OFFS)
    table5 = _relayout_tc(jnp.transpose(table, (1, 0)))
    g2 = _gather_sc(idx2, offs, table5)
    g3 = g2.reshape(L, B, 128)
    par = (jnp.transpose(xi, (1, 0)) >= HALF).astype(jnp.int32)
    par = par.reshape(L, 1, B)
    o = _select_transpose_tc(g3, par)
    return jnp.transpose(o, (2, 1, 0))


# speedup vs baseline: 1.6610x; 1.6610x over previous
import functools

import numpy as np

import jax
import jax.numpy as jnp
from jax import lax
from jax.experimental import pallas as pl
from jax.experimental.pallas import tpu as pltpu
from jax.experimental.pallas import tpu_sc as plsc

B, L, D = 4096, 50, 64
BL = B * L
NC, NS = 2, 16
NW = NC * NS
PER_W = BL // NW
CH = 128
NCH = PER_W // CH
RING = 5

_AR = np.arange(BL, dtype=np.int32)
_OFFS = ((_AR % L) * B + _AR // L).reshape(NW, NCH, CH)

_MESH = plsc.VectorSubcoreMesh(core_axis_name="c", subcore_axis_name="s")


@functools.partial(
    pl.kernel,
    mesh=_MESH,
    out_type=jax.ShapeDtypeStruct((BL, 128), jnp.float32),
    scratch_types=[
        pltpu.VMEM((NCH, CH), jnp.int32),
        pltpu.VMEM((NCH, CH), jnp.int32),
        pltpu.VMEM((RING, CH, 128), jnp.float32),
    ]
    + [pltpu.SemaphoreType.DMA] * (2 * RING),
)
def _gather_sc(idx_hbm, offs_hbm, table_hbm, g2_hbm, idx_v, offs_v, rows_v,
               *sems):
    gsems, ssems = sems[:RING], sems[RING:]
    wid = lax.axis_index("s") * NC + lax.axis_index("c")
    pltpu.sync_copy(idx_hbm.at[wid], idx_v)
    pltpu.sync_copy(offs_hbm.at[wid], offs_v)
    for b in range(RING):
        pltpu.async_copy(table_hbm.at[idx_v.at[b]], rows_v.at[b], gsems[b])

    def body(i, carry):
        for b in range(RING):
            j = i * RING + b
            pltpu.make_async_copy(
                table_hbm.at[idx_v.at[j]], rows_v.at[b], gsems[b]
            ).wait()
            pltpu.async_copy(rows_v.at[b], g2_hbm.at[offs_v.at[j]], ssems[b])

            @pl.when(i < NCH // RING - 1)
            def _():
                pltpu.make_async_copy(
                    rows_v.at[b], g2_hbm.at[offs_v.at[j]], ssems[b]
                ).wait()
                pltpu.async_copy(
                    table_hbm.at[idx_v.at[j + RING]], rows_v.at[b], gsems[b]
                )

        return carry

    lax.fori_loop(0, NCH // RING, body, 0)
    for b in range(RING):
        pltpu.make_async_copy(
            rows_v.at[b], g2_hbm.at[offs_v.at[NCH - RING + b]], ssems[b]
        ).wait()


CW = 4096
NT = 123
HALF = NT * CW
_CLAMP = (1000000 - 1) // CW


def _relayout_body(lo_ref, hi_ref, o_ref):
    t0 = jnp.swapaxes(lo_ref[...], 0, 1)
    t1 = jnp.swapaxes(hi_ref[...], 0, 1)
    o_ref[...] = jnp.concatenate([t0, t1], axis=1)[None]


def _relayout_tc(table_t):
    out = pl.pallas_call(
        _relayout_body,
        grid=(NT,),
        in_specs=[
            pl.BlockSpec((D, CW), lambda i: (0, i)),
            pl.BlockSpec((D, CW), lambda i: (0, jnp.minimum(NT + i, _CLAMP))),
        ],
        out_specs=pl.BlockSpec((1, CW, 128), lambda i: (i, 0, 0)),
        out_shape=jax.ShapeDtypeStruct((NT, CW, 128), jnp.float32),
    )(table_t, table_t)
    return out.reshape(HALF, 128)


NB = 4096


def _sel_tr_body(g_ref, par_ref, o_ref):
    gt = jnp.swapaxes(g_ref[...], 1, 2)
    par = par_ref[...]
    o_ref[...] = jnp.where(par == 1, gt[:, D:, :], gt[:, :D, :])


def _select_transpose_tc(g3, par):
    return pl.pallas_call(
        _sel_tr_body,
        grid=(L, B // NB),
        in_specs=[
            pl.BlockSpec((1, NB, 128), lambda l, i: (l, i, 0)),
            pl.BlockSpec((1, 1, NB), lambda l, i: (l, 0, i)),
        ],
        out_specs=pl.BlockSpec((1, D, NB), lambda l, i: (l, 0, i)),
        out_shape=jax.ShapeDtypeStruct((L, D, B), jnp.float32),
    )(g3, par)


def kernel(x, table):
    xi = x.astype(jnp.int32)
    idx2 = jnp.where(xi < HALF, xi, xi - HALF).reshape(NW, NCH, CH)
    offs = jnp.asarray(_OFFS)
    table5 = _relayout_tc(jnp.transpose(table, (1, 0)))
    g2 = _gather_sc(idx2, offs, table5)
    g3 = g2.reshape(L, B, 128)
    par = (jnp.transpose(xi, (1, 0)) >= HALF).astype(jnp.int32)
    par = par.reshape(L, 1, B)
    o = _select_transpose_tc(g3, par)
    return jnp.transpose(o, (2, 1, 0))


# speedup vs baseline: 1.7929x; 1.0794x over previous
import functools

import numpy as np

import jax
import jax.numpy as jnp
from jax import lax
from jax.experimental import pallas as pl
from jax.experimental.pallas import tpu as pltpu
from jax.experimental.pallas import tpu_sc as plsc

B, L, D = 4096, 50, 64
BL = B * L
NC, NS = 2, 16
NW = NC * NS
PER_W = BL // NW
CH = 128
NCH = PER_W // CH
RING = 5

_AR = np.arange(BL, dtype=np.int32)
_OFFS = ((_AR % L) * B + _AR // L).reshape(NW, NCH, CH)

_MESH = plsc.VectorSubcoreMesh(core_axis_name="c", subcore_axis_name="s")


@functools.partial(
    pl.kernel,
    mesh=_MESH,
    out_type=jax.ShapeDtypeStruct((BL, 128), jnp.float32),
    scratch_types=[
        pltpu.VMEM((NCH, CH), jnp.int32),
        pltpu.VMEM((NCH, CH), jnp.int32),
        pltpu.VMEM((RING, CH, 128), jnp.float32),
    ]
    + [pltpu.SemaphoreType.DMA] * (2 * RING),
)
def _gather_sc(idx_hbm, offs_hbm, table_hbm, g2_hbm, idx_v, offs_v, rows_v,
               *sems):
    gsems, ssems = sems[:RING], sems[RING:]
    wid = lax.axis_index("s") * NC + lax.axis_index("c")
    pltpu.sync_copy(idx_hbm.at[wid], idx_v)
    pltpu.sync_copy(offs_hbm.at[wid], offs_v)
    for b in range(RING):
        pltpu.async_copy(table_hbm.at[idx_v.at[b]], rows_v.at[b], gsems[b])

    def body(i, carry):
        for b in range(RING):
            j = i * RING + b
            pltpu.make_async_copy(
                table_hbm.at[idx_v.at[j]], rows_v.at[b], gsems[b]
            ).wait()
            pltpu.async_copy(rows_v.at[b], g2_hbm.at[offs_v.at[j]], ssems[b])

            @pl.when(i < NCH // RING - 1)
            def _():
                pltpu.make_async_copy(
                    rows_v.at[b], g2_hbm.at[offs_v.at[j]], ssems[b]
                ).wait()
                pltpu.async_copy(
                    table_hbm.at[idx_v.at[j + RING]], rows_v.at[b], gsems[b]
                )

        return carry

    lax.fori_loop(0, NCH // RING, body, 0)
    for b in range(RING):
        pltpu.make_async_copy(
            rows_v.at[b], g2_hbm.at[offs_v.at[NCH - RING + b]], ssems[b]
        ).wait()


CW = 8192
NT = 62
HALF = NT * CW
_CLAMP = (1000000 - 1) // CW


def _relayout_body(lo_ref, hi_ref, o_ref):
    t0 = jnp.swapaxes(lo_ref[...], 0, 1)
    t1 = jnp.swapaxes(hi_ref[...], 0, 1)
    o_ref[...] = jnp.concatenate([t0, t1], axis=1)[None]


def _relayout_tc(table_t):
    out = pl.pallas_call(
        _relayout_body,
        grid=(NT,),
        in_specs=[
            pl.BlockSpec((D, CW), lambda i: (0, i)),
            pl.BlockSpec((D, CW), lambda i: (0, jnp.minimum(NT + i, _CLAMP))),
        ],
        out_specs=pl.BlockSpec((1, CW, 128), lambda i: (i, 0, 0)),
        out_shape=jax.ShapeDtypeStruct((NT, CW, 128), jnp.float32),
    )(table_t, table_t)
    return out.reshape(HALF, 128)


NB = 4096


def _sel_tr_body(g_ref, par_ref, o_ref):
    gt = jnp.swapaxes(g_ref[...], 1, 2)
    par = par_ref[...]
    o_ref[...] = jnp.where(par == 1, gt[:, D:, :], gt[:, :D, :])


def _select_transpose_tc(g3, par):
    return pl.pallas_call(
        _sel_tr_body,
        grid=(L, B // NB),
        in_specs=[
            pl.BlockSpec((1, NB, 128), lambda l, i: (l, i, 0)),
            pl.BlockSpec((1, 1, NB), lambda l, i: (l, 0, i)),
        ],
        out_specs=pl.BlockSpec((1, D, NB), lambda l, i: (l, 0, i)),
        out_shape=jax.ShapeDtypeStruct((L, D, B), jnp.float32),
    )(g3, par)


def kernel(x, table):
    xi = x.astype(jnp.int32)
    idx2 = jnp.where(xi < HALF, xi, xi - HALF).reshape(NW, NCH, CH)
    offs = jnp.asarray(_OFFS)
    table5 = _relayout_tc(jnp.transpose(table, (1, 0)))
    g2 = _gather_sc(idx2, offs, table5)
    g3 = g2.reshape(L, B, 128)
    par = (jnp.transpose(xi, (1, 0)) >= HALF).astype(jnp.int32)
    par = par.reshape(L, 1, B)
    o = _select_transpose_tc(g3, par)
    return jnp.transpose(o, (2, 1, 0))


# speedup vs baseline: 1.8574x; 1.0359x over previous
import functools

import numpy as np

import jax
import jax.numpy as jnp
from jax import lax
from jax.experimental import pallas as pl
from jax.experimental.pallas import tpu as pltpu
from jax.experimental.pallas import tpu_sc as plsc

B, L, D = 4096, 50, 64
BL = B * L
NC, NS = 2, 16
NW = NC * NS
PER_W = BL // NW
CH = 128
NCH = PER_W // CH
RING = 5

_AR = np.arange(BL, dtype=np.int32)
_OFFS = ((_AR % L) * B + _AR // L).reshape(NW, NCH, CH)

_MESH = plsc.VectorSubcoreMesh(core_axis_name="c", subcore_axis_name="s")


@functools.partial(
    pl.kernel,
    mesh=_MESH,
    out_type=jax.ShapeDtypeStruct((BL, 128), jnp.float32),
    scratch_types=[
        pltpu.VMEM((NCH, CH), jnp.int32),
        pltpu.VMEM((NCH, CH), jnp.int32),
        pltpu.VMEM((RING, CH, 128), jnp.float32),
    ]
    + [pltpu.SemaphoreType.DMA] * (2 * RING),
)
def _gather_sc(idx_hbm, offs_hbm, table_hbm, g2_hbm, idx_v, offs_v, rows_v,
               *sems):
    gsems, ssems = sems[:RING], sems[RING:]
    wid = lax.axis_index("s") * NC + lax.axis_index("c")
    pltpu.sync_copy(idx_hbm.at[wid], idx_v)
    pltpu.sync_copy(offs_hbm.at[wid], offs_v)
    for b in range(RING):
        pltpu.async_copy(table_hbm.at[idx_v.at[b]], rows_v.at[b], gsems[b])

    def body(i, carry):
        for b in range(RING):
            j = i * RING + b
            pltpu.make_async_copy(
                table_hbm.at[idx_v.at[j]], rows_v.at[b], gsems[b]
            ).wait()
            pltpu.async_copy(rows_v.at[b], g2_hbm.at[offs_v.at[j]], ssems[b])

            @pl.when(i < NCH // RING - 1)
            def _():
                pltpu.make_async_copy(
                    rows_v.at[b], g2_hbm.at[offs_v.at[j]], ssems[b]
                ).wait()
                pltpu.async_copy(
                    table_hbm.at[idx_v.at[j + RING]], rows_v.at[b], gsems[b]
                )

        return carry

    lax.fori_loop(0, NCH // RING, body, 0)
    for b in range(RING):
        pltpu.make_async_copy(
            rows_v.at[b], g2_hbm.at[offs_v.at[NCH - RING + b]], ssems[b]
        ).wait()


CW = 16384
NT = 31
HALF = NT * CW
_CLAMP = (1000000 - 1) // CW


def _relayout_body(lo_ref, hi_ref, o_ref):
    t0 = jnp.swapaxes(lo_ref[...], 0, 1)
    t1 = jnp.swapaxes(hi_ref[...], 0, 1)
    o_ref[...] = jnp.concatenate([t0, t1], axis=1)[None]


def _relayout_tc(table_t):
    out = pl.pallas_call(
        _relayout_body,
        grid=(NT,),
        in_specs=[
            pl.BlockSpec((D, CW), lambda i: (0, i)),
            pl.BlockSpec((D, CW), lambda i: (0, jnp.minimum(NT + i, _CLAMP))),
        ],
        out_specs=pl.BlockSpec((1, CW, 128), lambda i: (i, 0, 0)),
        out_shape=jax.ShapeDtypeStruct((NT, CW, 128), jnp.float32),
    )(table_t, table_t)
    return out.reshape(HALF, 128)


NB = 4096


def _sel_tr_body(g_ref, par_ref, o_ref):
    gt = jnp.swapaxes(g_ref[...], 1, 2)
    par = par_ref[...]
    o_ref[...] = jnp.where(par == 1, gt[:, D:, :], gt[:, :D, :])


def _select_transpose_tc(g3, par):
    return pl.pallas_call(
        _sel_tr_body,
        grid=(L, B // NB),
        in_specs=[
            pl.BlockSpec((1, NB, 128), lambda l, i: (l, i, 0)),
            pl.BlockSpec((1, 1, NB), lambda l, i: (l, 0, i)),
        ],
        out_specs=pl.BlockSpec((1, D, NB), lambda l, i: (l, 0, i)),
        out_shape=jax.ShapeDtypeStruct((L, D, B), jnp.float32),
    )(g3, par)


def kernel(x, table):
    xi = x.astype(jnp.int32)
    idx2 = jnp.where(xi < HALF, xi, xi - HALF).reshape(NW, NCH, CH)
    offs = jnp.asarray(_OFFS)
    table5 = _relayout_tc(jnp.transpose(table, (1, 0)))
    g2 = _gather_sc(idx2, offs, table5)
    g3 = g2.reshape(L, B, 128)
    par = (jnp.transpose(xi, (1, 0)) >= HALF).astype(jnp.int32)
    par = par.reshape(L, 1, B)
    o = _select_transpose_tc(g3, par)
    return jnp.transpose(o, (2, 1, 0))
